# baseline (speedup 1.0000x reference)
import functools

import jax
import jax.numpy as jnp
from jax import lax
from jax.experimental import pallas as pl
from jax.experimental.pallas import tpu as pltpu
from jax.experimental.pallas import tpu_sc as plsc

NC = 2
NS = 16
K = 128
PH = 4
BN = 2000


def _mesh():
    return plsc.VectorSubcoreMesh(core_axis_name="c", subcore_axis_name="s")


def _make_deg_kernel(NP, CHT, RPT):
    @functools.partial(
        pl.kernel,
        out_type=jax.ShapeDtypeStruct((NP,), jnp.float32),
        mesh=_mesh(),
        scratch_types=[
            pltpu.VMEM((CHT, K), jnp.int32),
            pltpu.VMEM((K,), jnp.int32),
            pltpu.VMEM((K,), jnp.float32),
            pltpu.VMEM((RPT,), jnp.float32),
            pltpu.VMEM_SHARED((NP,), jnp.float32),
        ],
    )
    def deg_kernel(pidx_hbm, ones_hbm, out_hbm, pidx, didx, ones_v, vbuf,
                   acc_sh):
        c = lax.axis_index("c")
        s = lax.axis_index("s")
        row0 = s * RPT

        @pl.when(c == 0)
        def _():
            pltpu.sync_copy(pidx_hbm.at[s], pidx)
            pltpu.sync_copy(ones_hbm, ones_v)
            zsrc = jnp.zeros((16,), jnp.float32)

            def zfill(i, _):
                vbuf[pl.ds(i * 16, 16)] = zsrc
                return 0

            lax.fori_loop(0, RPT // 16, zfill, 0)
            pltpu.sync_copy(vbuf, acc_sh.at[pl.ds(row0, RPT)])
            plsc.subcore_barrier()

            def body(j, _):
                for i in range(K // 16):
                    p = pidx[j, pl.ds(i * 16, 16)]
                    didx[pl.ds(i * 16, 16)] = lax.shift_right_logical(p, 14)
                pltpu.sync_copy(ones_v, acc_sh.at[didx], add=True)
                return 0

            lax.fori_loop(0, CHT, body, 0)
            plsc.subcore_barrier()
            pltpu.sync_copy(acc_sh.at[pl.ds(row0, RPT)], vbuf)
            pltpu.sync_copy(vbuf, out_hbm.at[pl.ds(row0, RPT)])

    return deg_kernel


def _make_scatter_kernel(D, NP, CHT, RPT):
    CHP = CHT // PH

    @functools.partial(
        pl.kernel,
        out_type=jax.ShapeDtypeStruct((NP, D), jnp.float32),
        mesh=_mesh(),
        scratch_types=[
            pltpu.VMEM((CHP, K), jnp.int32),
            pltpu.VMEM((CHP, K), jnp.int32),
            pltpu.VMEM((K,), jnp.int32),
            pltpu.VMEM((K,), jnp.int32),
            pltpu.VMEM((K,), jnp.int32),
            pltpu.VMEM((K,), jnp.int32),
            pltpu.VMEM((K, D), jnp.float32),
            pltpu.VMEM((K, D), jnp.float32),
            pltpu.VMEM_SHARED((NP, D), jnp.float32),
            pltpu.SemaphoreType.DMA,
            pltpu.SemaphoreType.DMA,
            pltpu.SemaphoreType.DMA,
        ],
    )
    def scatter_kernel(g_hbm, pidx_hbm, zero_hbm, out_hbm,
                       pidx0, pidx1, sidx0, sidx1, didx0, didx1,
                       rows0, rows1, acc_sh, sem0, sem1, semp):
        c = lax.axis_index("c")
        s = lax.axis_index("s")
        row0 = s * RPT

        pbufs = (pidx0, pidx1)
        sbufs = (sidx0, sidx1)
        dbufs = (didx0, didx1)
        bufs = (rows0, rows1)
        sems = (sem0, sem1)

        @pl.when(c == 0)
        def _():
            pltpu.async_copy(pidx_hbm.at[s, pl.ds(0, CHP)], pidx0, semp)
            pltpu.sync_copy(zero_hbm, rows0)
            for t in range(RPT // K):
                pltpu.sync_copy(rows0, acc_sh.at[pl.ds(row0 + t * K, K), :])
            plsc.subcore_barrier()

            for p in range(PH):
                pidx = pbufs[p % 2]
                pltpu.make_async_copy(
                    pidx_hbm.at[s, pl.ds(0, CHP)], pidx, semp).wait()
                if p + 1 < PH:
                    pltpu.async_copy(
                        pidx_hbm.at[s, pl.ds((p + 1) * CHP, CHP)],
                        pbufs[(p + 1) % 2], semp)

                def unpack_src(j, b):
                    for i in range(K // 16):
                        q = pidx[j, pl.ds(i * 16, 16)]
                        sbufs[b][pl.ds(i * 16, 16)] = lax.bitwise_and(
                            q, 16383)

                def unpack_dst(j, b):
                    for i in range(K // 16):
                        q = pidx[j, pl.ds(i * 16, 16)]
                        dbufs[b][pl.ds(i * 16, 16)] = (
                            lax.shift_right_logical(q, 14))

                def fire(j, b):
                    pltpu.async_copy(g_hbm.at[sbufs[b]], bufs[b], sems[b])

                def drain(b):
                    pltpu.make_async_copy(zero_hbm, bufs[b], sems[b]).wait()

                def scat(b):
                    pltpu.sync_copy(bufs[b], acc_sh.at[dbufs[b]], add=True)

                unpack_src(0, 0)
                fire(0, 0)
                unpack_src(1, 1)
                fire(1, 1)
                unpack_dst(0, 0)
                unpack_dst(1, 1)

                def body(jj, _):
                    j = 2 * jj
                    drain(0)

                    @pl.when(j + 2 < CHP)
                    def _():
                        unpack_src(j + 2, 0)
                        fire(j + 2, 0)

                    scat(0)

                    @pl.when(j + 2 < CHP)
                    def _():
                        unpack_dst(j + 2, 0)

                    drain(1)

                    @pl.when(j + 3 < CHP)
                    def _():
                        unpack_src(j + 3, 1)
                        fire(j + 3, 1)

                    scat(1)

                    @pl.when(j + 3 < CHP)
                    def _():
                        unpack_dst(j + 3, 1)

                    return 0

                lax.fori_loop(0, CHP // 2, body, 0)

            plsc.subcore_barrier()
            for t in range(RPT // K):
                pltpu.sync_copy(acc_sh.at[pl.ds(row0 + t * K, K), :], rows0)
                pltpu.sync_copy(rows0, out_hbm.at[pl.ds(row0 + t * K, K), :])

    return scatter_kernel


def _stage1_body(deg_ref, x_ref, w1_ref, g1_ref, dis_ref):
    dis = lax.rsqrt(deg_ref[...] + 1.0)
    h = jnp.dot(x_ref[...], w1_ref[...], preferred_element_type=jnp.float32)
    g1_ref[...] = dis * h
    dis_ref[...] = dis


def _stage2_body(part_ref, g1_ref, dis_ref, b1_ref, w2_ref, g2_ref):
    dis = dis_ref[...]
    p = part_ref[...] + g1_ref[...]
    h = jnp.maximum(dis * p + b1_ref[...], 0.0)
    g2_ref[...] = dis * jnp.dot(h, w2_ref[...],
                                preferred_element_type=jnp.float32)


def _stage3_body(D_out, part_ref, g2_ref, dis_ref, b2_ref, out_ref):
    p = (part_ref[...] + g2_ref[...])[:, :D_out]
    t = dis_ref[...] * p + b2_ref[...]
    m = jnp.max(t, axis=1, keepdims=True)
    e = t - m
    out_ref[...] = e - jnp.log(jnp.sum(jnp.exp(e), axis=1, keepdims=True))


def kernel(x, edge_index, W1, b1, W2, b2):
    N, D_in = x.shape
    E = edge_index.shape[1]
    D_hid = W1.shape[1]
    D_out = W2.shape[1]

    CHT = -(-E // (NS * K * PH)) * PH
    EP = NS * CHT * K
    RPT = -(-(N + 1) // (NS * 128)) * 128
    NP = NS * RPT

    src = edge_index[0].astype(jnp.int32)
    dst = edge_index[1].astype(jnp.int32)
    pad = EP - E
    src_p = jnp.concatenate([src, jnp.zeros((pad,), jnp.int32)])
    dst_p = jnp.concatenate([dst, jnp.full((pad,), N, jnp.int32)])
    pidx3 = (dst_p * 16384 + src_p).reshape(NS, CHT, K)

    ones_k = jnp.ones((K,), jnp.float32)
    zeros_hid = jnp.zeros((K, D_hid), jnp.float32)
    D2 = 128
    W2p = jnp.zeros((D_hid, D2), jnp.float32).at[:, :D_out].set(W2)

    deg = _make_deg_kernel(NP, CHT, RPT)(pidx3, ones_k)

    nb = N // BN
    assert N % BN == 0

    g1, dis = pl.pallas_call(
        _stage1_body,
        grid=(nb,),
        in_specs=[
            pl.BlockSpec((BN, 1), lambda i: (i, 0)),
            pl.BlockSpec((BN, D_in), lambda i: (i, 0)),
            pl.BlockSpec((D_in, D_hid), lambda i: (0, 0)),
        ],
        out_specs=[
            pl.BlockSpec((BN, D_hid), lambda i: (i, 0)),
            pl.BlockSpec((BN, 1), lambda i: (i, 0)),
        ],
        out_shape=[
            jax.ShapeDtypeStruct((N, D_hid), jnp.float32),
            jax.ShapeDtypeStruct((N, 1), jnp.float32),
        ],
    )(deg.reshape(NP, 1), x, W1)

    part1 = _make_scatter_kernel(D_hid, NP, CHT, RPT)(g1, pidx3, zeros_hid)

    g2 = pl.pallas_call(
        _stage2_body,
        grid=(nb,),
        in_specs=[
            pl.BlockSpec((BN, D_hid), lambda i: (i, 0)),
            pl.BlockSpec((BN, D_hid), lambda i: (i, 0)),
            pl.BlockSpec((BN, 1), lambda i: (i, 0)),
            pl.BlockSpec((1, D_hid), lambda i: (0, 0)),
            pl.BlockSpec((D_hid, D2), lambda i: (0, 0)),
        ],
        out_specs=pl.BlockSpec((BN, D2), lambda i: (i, 0)),
        out_shape=jax.ShapeDtypeStruct((N, D2), jnp.float32),
    )(part1, g1, dis, b1.reshape(1, D_hid), W2p)

    part2 = _make_scatter_kernel(D2, NP, CHT, RPT)(g2, pidx3, zeros_hid)

    out = pl.pallas_call(
        functools.partial(_stage3_body, D_out),
        grid=(nb,),
        in_specs=[
            pl.BlockSpec((BN, D2), lambda i: (i, 0)),
            pl.BlockSpec((BN, D2), lambda i: (i, 0)),
            pl.BlockSpec((BN, 1), lambda i: (i, 0)),
            pl.BlockSpec((1, D_out), lambda i: (0, 0)),
        ],
        out_specs=pl.BlockSpec((BN, D_out), lambda i: (i, 0)),
        out_shape=jax.ShapeDtypeStruct((N, D_out), jnp.float32),
    )(part2, g2, dis, b2.reshape(1, D_out))

    return out


# speedup vs baseline: 1.7312x; 1.7312x over previous
import functools

import jax
import jax.numpy as jnp
from jax import lax
from jax.experimental import pallas as pl
from jax.experimental.pallas import tpu as pltpu
from jax.experimental.pallas import tpu_sc as plsc

NC = 2
NS = 16
K = 128
PH = 4
BN = 2000


def _mesh():
    return plsc.VectorSubcoreMesh(core_axis_name="c", subcore_axis_name="s")


def _make_deg_kernel(NP, CH, RPT):
    @functools.partial(
        pl.kernel,
        out_type=jax.ShapeDtypeStruct((NC, NP), jnp.float32),
        mesh=_mesh(),
        scratch_types=[
            pltpu.VMEM((CH, K), jnp.int32),
            pltpu.VMEM((K,), jnp.int32),
            pltpu.VMEM((K,), jnp.float32),
            pltpu.VMEM((RPT,), jnp.float32),
            pltpu.VMEM_SHARED((NP,), jnp.float32),
        ],
    )
    def deg_kernel(pidx_hbm, ones_hbm, out_hbm, pidx, didx, ones_v, vbuf,
                   acc_sh):
        c = lax.axis_index("c")
        s = lax.axis_index("s")
        w = s * NC + c
        row0 = s * RPT
        pltpu.sync_copy(pidx_hbm.at[w], pidx)
        pltpu.sync_copy(ones_hbm, ones_v)
        zsrc = jnp.zeros((16,), jnp.float32)

        def zfill(i, _):
            vbuf[pl.ds(i * 16, 16)] = zsrc
            return 0

        lax.fori_loop(0, RPT // 16, zfill, 0)
        pltpu.sync_copy(vbuf, acc_sh.at[pl.ds(row0, RPT)])
        plsc.subcore_barrier()

        def body(j, _):
            for i in range(K // 16):
                p = pidx[j, pl.ds(i * 16, 16)]
                didx[pl.ds(i * 16, 16)] = lax.shift_right_logical(p, 14)
            pltpu.sync_copy(ones_v, acc_sh.at[didx], add=True)
            return 0

        lax.fori_loop(0, CH, body, 0)
        plsc.subcore_barrier()
        pltpu.sync_copy(acc_sh.at[pl.ds(row0, RPT)], vbuf)
        pltpu.sync_copy(vbuf, out_hbm.at[c, pl.ds(row0, RPT)])

    return deg_kernel


def _make_scatter_kernel(D, NP, CH, RPT):
    @functools.partial(
        pl.kernel,
        out_type=jax.ShapeDtypeStruct((NC, NP, D), jnp.float32),
        mesh=_mesh(),
        scratch_types=[
            pltpu.VMEM((CH, K), jnp.int32),
            pltpu.VMEM((K,), jnp.int32),
            pltpu.VMEM((K,), jnp.int32),
            pltpu.VMEM((K,), jnp.int32),
            pltpu.VMEM((K,), jnp.int32),
            pltpu.VMEM((K, D), jnp.float32),
            pltpu.VMEM((K, D), jnp.float32),
            pltpu.VMEM_SHARED((NP, D), jnp.float32),
            pltpu.SemaphoreType.DMA,
            pltpu.SemaphoreType.DMA,
        ],
    )
    def scatter_kernel(g_hbm, pidx_hbm, zero_hbm, out_hbm,
                       pidx, sidx0, sidx1, didx0, didx1,
                       rows0, rows1, acc_sh, sem0, sem1):
        c = lax.axis_index("c")
        s = lax.axis_index("s")
        w = s * NC + c
        row0 = s * RPT

        sbufs = (sidx0, sidx1)
        dbufs = (didx0, didx1)
        bufs = (rows0, rows1)
        sems = (sem0, sem1)

        pltpu.sync_copy(pidx_hbm.at[w], pidx)
        pltpu.sync_copy(zero_hbm, rows0)
        for t in range(RPT // K):
            pltpu.sync_copy(rows0, acc_sh.at[pl.ds(row0 + t * K, K), :])
        plsc.subcore_barrier()

        def unpack_src(j, b):
            for i in range(K // 16):
                q = pidx[j, pl.ds(i * 16, 16)]
                sbufs[b][pl.ds(i * 16, 16)] = lax.bitwise_and(q, 16383)

        def unpack_dst(j, b):
            for i in range(K // 16):
                q = pidx[j, pl.ds(i * 16, 16)]
                dbufs[b][pl.ds(i * 16, 16)] = lax.shift_right_logical(q, 14)

        def fire(j, b):
            pltpu.async_copy(g_hbm.at[sbufs[b]], bufs[b], sems[b])

        def drain(b):
            pltpu.make_async_copy(zero_hbm, bufs[b], sems[b]).wait()

        def scat(b):
            pltpu.sync_copy(bufs[b], acc_sh.at[dbufs[b]], add=True)

        unpack_src(0, 0)
        fire(0, 0)
        unpack_src(1, 1)
        fire(1, 1)
        unpack_dst(0, 0)
        unpack_dst(1, 1)

        def body(jj, _):
            j = 2 * jj
            drain(0)

            @pl.when(j + 2 < CH)
            def _():
                unpack_src(j + 2, 0)
                fire(j + 2, 0)

            scat(0)

            @pl.when(j + 2 < CH)
            def _():
                unpack_dst(j + 2, 0)

            drain(1)

            @pl.when(j + 3 < CH)
            def _():
                unpack_src(j + 3, 1)
                fire(j + 3, 1)

            scat(1)

            @pl.when(j + 3 < CH)
            def _():
                unpack_dst(j + 3, 1)

            return 0

        lax.fori_loop(0, CH // 2, body, 0)
        if CH % 2:
            drain(0)
            scat(0)
        plsc.subcore_barrier()
        for t in range(RPT // K):
            pltpu.sync_copy(acc_sh.at[pl.ds(row0 + t * K, K), :], rows0)
            pltpu.sync_copy(rows0, out_hbm.at[c, pl.ds(row0 + t * K, K), :])

    return scatter_kernel


def _stage1_body(deg_ref, x_ref, w1_ref, g1_ref, dis_ref):
    dis = lax.rsqrt(deg_ref[:, 0:1] + deg_ref[:, 1:2] + 1.0)
    h = jnp.dot(x_ref[...], w1_ref[...], preferred_element_type=jnp.float32)
    g1_ref[...] = dis * h
    dis_ref[...] = dis


def _stage2_body(part_ref, g1_ref, dis_ref, b1_ref, w2_ref, g2_ref):
    dis = dis_ref[...]
    p = part_ref[0] + part_ref[1] + g1_ref[...]
    h = jnp.maximum(dis * p + b1_ref[...], 0.0)
    g2_ref[...] = dis * jnp.dot(h, w2_ref[...],
                                preferred_element_type=jnp.float32)


def _stage3_body(D_out, part_ref, g2_ref, dis_ref, b2_ref, out_ref):
    p = (part_ref[0] + part_ref[1] + g2_ref[...])[:, :D_out]
    t = dis_ref[...] * p + b2_ref[...]
    m = jnp.max(t, axis=1, keepdims=True)
    e = t - m
    out_ref[...] = e - jnp.log(jnp.sum(jnp.exp(e), axis=1, keepdims=True))


def kernel(x, edge_index, W1, b1, W2, b2):
    N, D_in = x.shape
    E = edge_index.shape[1]
    D_hid = W1.shape[1]
    D_out = W2.shape[1]

    NW = NC * NS
    CH = -(-E // (NW * K))
    EP = NW * CH * K
    RPT = -(-(N + 1) // (NS * 128)) * 128
    NP = NS * RPT

    src = edge_index[0].astype(jnp.int32)
    dst = edge_index[1].astype(jnp.int32)
    pad = EP - E
    src_p = jnp.concatenate([src, jnp.zeros((pad,), jnp.int32)])
    dst_p = jnp.concatenate([dst, jnp.full((pad,), N, jnp.int32)])
    pidx3 = (dst_p * 16384 + src_p).reshape(NW, CH, K)

    ones_k = jnp.ones((K,), jnp.float32)
    zeros_hid = jnp.zeros((K, D_hid), jnp.float32)
    D2 = 128
    W2p = jnp.zeros((D_hid, D2), jnp.float32).at[:, :D_out].set(W2)

    deg_part = _make_deg_kernel(NP, CH, RPT)(pidx3, ones_k)
    deg_t = deg_part.T

    nb = N // BN
    assert N % BN == 0

    g1, dis = pl.pallas_call(
        _stage1_body,
        grid=(nb,),
        in_specs=[
            pl.BlockSpec((BN, NC), lambda i: (i, 0)),
            pl.BlockSpec((BN, D_in), lambda i: (i, 0)),
            pl.BlockSpec((D_in, D_hid), lambda i: (0, 0)),
        ],
        out_specs=[
            pl.BlockSpec((BN, D_hid), lambda i: (i, 0)),
            pl.BlockSpec((BN, 1), lambda i: (i, 0)),
        ],
        out_shape=[
            jax.ShapeDtypeStruct((N, D_hid), jnp.float32),
            jax.ShapeDtypeStruct((N, 1), jnp.float32),
        ],
    )(deg_t, x, W1)

    part1 = _make_scatter_kernel(D_hid, NP, CH, RPT)(g1, pidx3, zeros_hid)

    g2 = pl.pallas_call(
        _stage2_body,
        grid=(nb,),
        in_specs=[
            pl.BlockSpec((NC, BN, D_hid), lambda i: (0, i, 0)),
            pl.BlockSpec((BN, D_hid), lambda i: (i, 0)),
            pl.BlockSpec((BN, 1), lambda i: (i, 0)),
            pl.BlockSpec((1, D_hid), lambda i: (0, 0)),
            pl.BlockSpec((D_hid, D2), lambda i: (0, 0)),
        ],
        out_specs=pl.BlockSpec((BN, D2), lambda i: (i, 0)),
        out_shape=jax.ShapeDtypeStruct((N, D2), jnp.float32),
    )(part1, g1, dis, b1.reshape(1, D_hid), W2p)

    part2 = _make_scatter_kernel(D2, NP, CH, RPT)(g2, pidx3, zeros_hid)

    out = pl.pallas_call(
        functools.partial(_stage3_body, D_out),
        grid=(nb,),
        in_specs=[
            pl.BlockSpec((NC, BN, D2), lambda i: (0, i, 0)),
            pl.BlockSpec((BN, D2), lambda i: (i, 0)),
            pl.BlockSpec((BN, 1), lambda i: (i, 0)),
            pl.BlockSpec((1, D_out), lambda i: (0, 0)),
        ],
        out_specs=pl.BlockSpec((BN, D_out), lambda i: (i, 0)),
        out_shape=jax.ShapeDtypeStruct((N, D_out), jnp.float32),
    )(part2, g2, dis, b2.reshape(1, D_out))

    return out


# speedup vs baseline: 2.3972x; 1.3847x over previous
import functools

import jax
import jax.numpy as jnp
from jax import lax
from jax.experimental import pallas as pl
from jax.experimental.pallas import tpu as pltpu
from jax.experimental.pallas import tpu_sc as plsc

NC = 2
NS = 16
K = 128
PH = 4
BN = 2000


def _mesh():
    return plsc.VectorSubcoreMesh(core_axis_name="c", subcore_axis_name="s")


def _make_deg_kernel(NP, CH, RPT):
    @functools.partial(
        pl.kernel,
        out_type=jax.ShapeDtypeStruct((NC, NP), jnp.float32),
        mesh=_mesh(),
        scratch_types=[
            pltpu.VMEM((CH, K), jnp.int32),
            pltpu.VMEM((K,), jnp.int32),
            pltpu.VMEM((K,), jnp.float32),
            pltpu.VMEM((RPT,), jnp.float32),
            pltpu.VMEM_SHARED((NP,), jnp.float32),
        ],
    )
    def deg_kernel(pidx_hbm, ones_hbm, out_hbm, pidx, didx, ones_v, vbuf,
                   acc_sh):
        c = lax.axis_index("c")
        s = lax.axis_index("s")
        w = s * NC + c
        row0 = s * RPT
        pltpu.sync_copy(pidx_hbm.at[w], pidx)
        pltpu.sync_copy(ones_hbm, ones_v)
        zsrc = jnp.zeros((16,), jnp.float32)

        def zfill(i, _):
            vbuf[pl.ds(i * 16, 16)] = zsrc
            return 0

        lax.fori_loop(0, RPT // 16, zfill, 0)
        pltpu.sync_copy(vbuf, acc_sh.at[pl.ds(row0, RPT)])
        plsc.subcore_barrier()

        def body(j, _):
            for i in range(K // 16):
                p = pidx[j, pl.ds(i * 16, 16)]
                didx[pl.ds(i * 16, 16)] = lax.shift_right_logical(p, 14)
            pltpu.sync_copy(ones_v, acc_sh.at[didx], add=True)
            return 0

        lax.fori_loop(0, CH, body, 0)
        plsc.subcore_barrier()
        pltpu.sync_copy(acc_sh.at[pl.ds(row0, RPT)], vbuf)
        pltpu.sync_copy(vbuf, out_hbm.at[c, pl.ds(row0, RPT)])

    return deg_kernel


def _make_scatter_kernel(D, NP, CH, RPT, untiled=False):
    @functools.partial(
        pl.kernel,
        out_type=jax.ShapeDtypeStruct((NC, NP, D), jnp.float32),
        mesh=_mesh(),
        compiler_params=pltpu.CompilerParams(use_tc_tiling_on_sc=False)
        if untiled else None,
        scratch_types=[
            pltpu.VMEM((CH, K), jnp.int32),
            pltpu.VMEM((K,), jnp.int32),
            pltpu.VMEM((K,), jnp.int32),
            pltpu.VMEM((K,), jnp.int32),
            pltpu.VMEM((K,), jnp.int32),
            pltpu.VMEM((K, D), jnp.float32),
            pltpu.VMEM((K, D), jnp.float32),
            pltpu.VMEM_SHARED((NP, D), jnp.float32),
            pltpu.SemaphoreType.DMA,
            pltpu.SemaphoreType.DMA,
        ],
    )
    def scatter_kernel(g_hbm, pidx_hbm, zero_hbm, out_hbm,
                       pidx, sidx0, sidx1, didx0, didx1,
                       rows0, rows1, acc_sh, sem0, sem1):
        c = lax.axis_index("c")
        s = lax.axis_index("s")
        w = s * NC + c
        row0 = s * RPT

        sbufs = (sidx0, sidx1)
        dbufs = (didx0, didx1)
        bufs = (rows0, rows1)
        sems = (sem0, sem1)

        pltpu.sync_copy(pidx_hbm.at[w], pidx)
        pltpu.sync_copy(zero_hbm, rows0)
        for t in range(RPT // K):
            pltpu.sync_copy(rows0, acc_sh.at[pl.ds(row0 + t * K, K), :])
        plsc.subcore_barrier()

        def unpack_src(j, b):
            for i in range(K // 16):
                q = pidx[j, pl.ds(i * 16, 16)]
                sbufs[b][pl.ds(i * 16, 16)] = lax.bitwise_and(q, 16383)

        def unpack_dst(j, b):
            for i in range(K // 16):
                q = pidx[j, pl.ds(i * 16, 16)]
                dbufs[b][pl.ds(i * 16, 16)] = lax.shift_right_logical(q, 14)

        def fire(j, b):
            pltpu.async_copy(g_hbm.at[sbufs[b]], bufs[b], sems[b])

        def drain(b):
            pltpu.make_async_copy(zero_hbm, bufs[b], sems[b]).wait()

        def scat(b):
            pltpu.sync_copy(bufs[b], acc_sh.at[dbufs[b]], add=True)

        unpack_src(0, 0)
        fire(0, 0)
        unpack_src(1, 1)
        fire(1, 1)
        unpack_dst(0, 0)
        unpack_dst(1, 1)

        def body(jj, _):
            j = 2 * jj
            drain(0)

            @pl.when(j + 2 < CH)
            def _():
                unpack_src(j + 2, 0)
                fire(j + 2, 0)

            scat(0)

            @pl.when(j + 2 < CH)
            def _():
                unpack_dst(j + 2, 0)

            drain(1)

            @pl.when(j + 3 < CH)
            def _():
                unpack_src(j + 3, 1)
                fire(j + 3, 1)

            scat(1)

            @pl.when(j + 3 < CH)
            def _():
                unpack_dst(j + 3, 1)

            return 0

        lax.fori_loop(0, CH // 2, body, 0)
        if CH % 2:
            drain(0)
            scat(0)
        plsc.subcore_barrier()
        for t in range(RPT // K):
            pltpu.sync_copy(acc_sh.at[pl.ds(row0 + t * K, K), :], rows0)
            pltpu.sync_copy(rows0, out_hbm.at[c, pl.ds(row0 + t * K, K), :])

    return scatter_kernel


def _stage1_body(deg_ref, x_ref, w1_ref, g1_ref, dis_ref):
    dis = lax.rsqrt(deg_ref[:, 0:1] + deg_ref[:, 1:2] + 1.0)
    h = jnp.dot(x_ref[...], w1_ref[...], preferred_element_type=jnp.float32)
    g1_ref[...] = dis * h
    dis_ref[...] = dis


def _stage2_body(part_ref, g1_ref, dis_ref, b1_ref, w2_ref, g2_ref):
    dis = dis_ref[...]
    p = part_ref[0] + part_ref[1] + g1_ref[...]
    h = jnp.maximum(dis * p + b1_ref[...], 0.0)
    g2_ref[...] = dis * jnp.dot(h, w2_ref[...],
                                preferred_element_type=jnp.float32)


def _stage3_body(D_out, part_ref, g2_ref, dis_ref, b2_ref, out_ref):
    p = part_ref[0] + part_ref[1] + g2_ref[...]
    t = dis_ref[...] * p + b2_ref[...]
    m = jnp.max(t, axis=1, keepdims=True)
    e = t - m
    out_ref[...] = e - jnp.log(jnp.sum(jnp.exp(e), axis=1, keepdims=True))


def kernel(x, edge_index, W1, b1, W2, b2):
    N, D_in = x.shape
    E = edge_index.shape[1]
    D_hid = W1.shape[1]
    D_out = W2.shape[1]

    NW = NC * NS
    CH = -(-E // (NW * K))
    EP = NW * CH * K
    RPT = -(-(N + 1) // (NS * 128)) * 128
    NP = NS * RPT

    src = edge_index[0].astype(jnp.int32)
    dst = edge_index[1].astype(jnp.int32)
    pad = EP - E
    src_p = jnp.concatenate([src, jnp.zeros((pad,), jnp.int32)])
    dst_p = jnp.concatenate([dst, jnp.full((pad,), N, jnp.int32)])
    pidx3 = (dst_p * 16384 + src_p).reshape(NW, CH, K)

    ones_k = jnp.ones((K,), jnp.float32)
    zeros_hid = jnp.zeros((K, D_hid), jnp.float32)
    zeros_out = jnp.zeros((K, D_out), jnp.float32)
    D2 = D_out

    deg_part = _make_deg_kernel(NP, CH, RPT)(pidx3, ones_k)
    deg_t = deg_part.T

    nb = N // BN
    assert N % BN == 0

    g1, dis = pl.pallas_call(
        _stage1_body,
        grid=(nb,),
        in_specs=[
            pl.BlockSpec((BN, NC), lambda i: (i, 0)),
            pl.BlockSpec((BN, D_in), lambda i: (i, 0)),
            pl.BlockSpec((D_in, D_hid), lambda i: (0, 0)),
        ],
        out_specs=[
            pl.BlockSpec((BN, D_hid), lambda i: (i, 0)),
            pl.BlockSpec((BN, 1), lambda i: (i, 0)),
        ],
        out_shape=[
            jax.ShapeDtypeStruct((N, D_hid), jnp.float32),
            jax.ShapeDtypeStruct((N, 1), jnp.float32),
        ],
    )(deg_t, x, W1)

    part1 = _make_scatter_kernel(D_hid, NP, CH, RPT)(g1, pidx3, zeros_hid)

    g2 = pl.pallas_call(
        _stage2_body,
        grid=(nb,),
        in_specs=[
            pl.BlockSpec((NC, BN, D_hid), lambda i: (0, i, 0)),
            pl.BlockSpec((BN, D_hid), lambda i: (i, 0)),
            pl.BlockSpec((BN, 1), lambda i: (i, 0)),
            pl.BlockSpec((1, D_hid), lambda i: (0, 0)),
            pl.BlockSpec((D_hid, D2), lambda i: (0, 0)),
        ],
        out_specs=pl.BlockSpec((BN, D2), lambda i: (i, 0)),
        out_shape=jax.ShapeDtypeStruct((N, D2), jnp.float32),
    )(part1, g1, dis, b1.reshape(1, D_hid), W2)

    part2 = _make_scatter_kernel(D2, NP, CH, RPT, untiled=True)(
        g2, pidx3, zeros_out)

    out = pl.pallas_call(
        functools.partial(_stage3_body, D_out),
        grid=(nb,),
        in_specs=[
            pl.BlockSpec((NC, BN, D2), lambda i: (0, i, 0)),
            pl.BlockSpec((BN, D2), lambda i: (i, 0)),
            pl.BlockSpec((BN, 1), lambda i: (i, 0)),
            pl.BlockSpec((1, D_out), lambda i: (0, 0)),
        ],
        out_specs=pl.BlockSpec((BN, D_out), lambda i: (i, 0)),
        out_shape=jax.ShapeDtypeStruct((N, D_out), jnp.float32),
    )(part2, g2, dis, b2.reshape(1, D_out))

    return out
